# baseline (speedup 1.0000x reference)
import functools

import jax
import jax.numpy as jnp
from jax import lax
from jax.experimental import pallas as pl
from jax.experimental.pallas import tpu as pltpu
from jax.experimental.pallas import tpu_sc as plsc

_BIG_KEY = 0x7F7FFFFF
_MASK12 = ~4095


def _level_body(win_ref, pu_ref, bu_ref, posb_ref, bcb_ref, xb_ref, xs_ref,
                w_top_ref, w_bot_ref, b_up_ref, wa_ref, ba_ref,
                wb_ref, bb_ref, out_ref, m_ref, key_ref, up_ref,
                *, n_blocks, block_c):
    R = pu_ref.shape[0]
    jlo = win_ref[0, 0, 0]
    jhi = win_ref[0, 0, 1]

    LW = 128
    m_ref[...] = jnp.full((R, 3 * LW), _BIG_KEY, jnp.int32)
    up_ref[...] = jnp.zeros_like(up_ref)

    bu = bu_ref[...]
    lane = lax.broadcasted_iota(jnp.int32, (R, block_c), 1)

    def sweep1(j, _):
        @pl.when(jnp.logical_and(j >= jlo, j < jhi))
        def _():
            d2 = jnp.zeros((R, block_c), jnp.float32)
            for c in range(3):
                diff = pu_ref[:, c:c + 1] - posb_ref[j, c:c + 1, :]
                d2 = d2 + diff * diff
            key = (lax.bitcast_convert_type(d2, jnp.int32) & _MASK12) \
                | (lane + j * block_c)
            key = jnp.where(bu == bcb_ref[j], key, _BIG_KEY)
            key_ref[j] = key
            u1 = m_ref[:, 0 * LW:1 * LW]
            u2 = m_ref[:, 1 * LW:2 * LW]
            u3 = m_ref[:, 2 * LW:3 * LW]
            for s in range(block_c // LW):
                v = key[:, s * LW:(s + 1) * LW]
                t1 = jnp.maximum(u1, v)
                u1 = jnp.minimum(u1, v)
                t2 = jnp.maximum(u2, t1)
                u2 = jnp.minimum(u2, t1)
                u3 = jnp.minimum(u3, t2)
            m_ref[:, 0 * LW:1 * LW] = u1
            m_ref[:, 1 * LW:2 * LW] = u2
            m_ref[:, 2 * LW:3 * LW] = u3
        return 0

    lax.fori_loop(0, n_blocks, sweep1, 0, unroll=True)

    u1 = m_ref[:, 0 * LW:1 * LW]
    u2 = m_ref[:, 1 * LW:2 * LW]
    u3 = m_ref[:, 2 * LW:3 * LW]
    ms = []
    for _ in range(3):
        mk = jnp.min(u1, axis=1, keepdims=True)
        sel = u1 == mk
        u1 = jnp.where(sel, u2, u1)
        u2 = jnp.where(sel, u3, u2)
        u3 = jnp.where(sel, _BIG_KEY, u3)
        ms.append(mk)
    m3 = ms[2]

    def sweep2(j, _):
        @pl.when(jnp.logical_and(j >= jlo, j < jhi))
        def _():
            key = key_ref[j]
            d2q = lax.bitcast_convert_type(key & _MASK12, jnp.float32)
            w = 1.0 / jnp.maximum(d2q, 1e-16)
            wmat = jnp.where(key <= m3, w, 0.0)
            up_ref[...] += jnp.dot(wmat, xb_ref[j],
                                   preferred_element_type=jnp.float32)
        return 0

    lax.fori_loop(0, n_blocks, sweep2, 0, unroll=True)

    den = jnp.zeros((R, 1), jnp.float32)
    for mk in ms:
        dk = lax.bitcast_convert_type(mk & _MASK12, jnp.float32)
        den = den + 1.0 / jnp.maximum(dk, 1e-16)
    up = up_ref[...] / den

    xi = jnp.dot(xs_ref[...], w_top_ref[...], preferred_element_type=jnp.float32)
    xi = xi + jnp.dot(up, w_bot_ref[...], preferred_element_type=jnp.float32)
    xi = jax.nn.relu(xi + b_up_ref[...])
    h = jax.nn.relu(jnp.dot(xi, wa_ref[...], preferred_element_type=jnp.float32)
                    + ba_ref[...])
    out_ref[...] = xi + jnp.dot(h, wb_ref[...],
                                preferred_element_type=jnp.float32) + bb_ref[...]


def _level(pos_up, batch_up, pos, batch, x, x_skip,
           W_up, b_up, Wa, ba, Wb, bb, row_tile, block_c):
    n_up = pos_up.shape[0]
    n_coarse = pos.shape[0]
    c_skip = x_skip.shape[1]
    c_out = Wa.shape[0]
    n_blocks = n_coarse // block_c
    n_tiles = n_up // row_tile

    bu_i = batch_up.reshape(n_tiles, row_tile)
    lo = jnp.searchsorted(batch, bu_i[:, 0], side="left")
    hi = jnp.searchsorted(batch, bu_i[:, -1], side="right")
    wins = jnp.stack([lo // block_c,
                      (hi + block_c - 1) // block_c], axis=1).astype(jnp.int32)

    posb = pos.T.reshape(3, n_blocks, block_c).transpose(1, 0, 2)
    bcb = batch.astype(jnp.int32).reshape(n_blocks, 1, block_c)
    xb = x.reshape(n_blocks, block_c, x.shape[1])
    bu2 = batch_up.astype(jnp.int32)[:, None]
    w_top = W_up[:c_skip]
    w_bot = W_up[c_skip:]

    row_spec = lambda cols_: pl.BlockSpec((row_tile, cols_), lambda i: (i, 0))
    full = lambda a: pl.BlockSpec(a.shape, lambda i: (0,) * a.ndim)

    fn = pl.pallas_call(
        functools.partial(_level_body, n_blocks=n_blocks, block_c=block_c),
        grid=(n_tiles,),
        in_specs=[
            pl.BlockSpec((1, 1, 2), lambda i: (i, 0, 0),
                         memory_space=pltpu.SMEM),
            row_spec(3),
            row_spec(1),
            full(posb),
            full(bcb),
            full(xb),
            row_spec(c_skip),
            full(w_top), full(w_bot), full(b_up[None, :]),
            full(Wa), full(ba[None, :]),
            full(Wb), full(bb[None, :]),
        ],
        out_specs=row_spec(c_out),
        out_shape=jax.ShapeDtypeStruct((n_up, c_out), jnp.float32),
        scratch_shapes=[
            pltpu.VMEM((row_tile, 3 * 128), jnp.int32),
            pltpu.VMEM((n_blocks, row_tile, block_c), jnp.int32),
            pltpu.VMEM((row_tile, x.shape[1]), jnp.float32),
        ],
    )
    return fn(wins[:, None, :], pos_up, bu2, posb, bcb, xb, x_skip,
              w_top, w_bot, b_up[None, :], Wa, ba[None, :], Wb, bb[None, :])


def _knn_body(win_ref, pu_ref, bu_ref, posb_ref, bcb_ref,
              idx_ref, w_ref, m_ref, *, n_blocks, block_c):
    R = pu_ref.shape[0]
    jlo = win_ref[0, 0, 0]
    jhi = win_ref[0, 0, 1]

    LW = 128
    m_ref[...] = jnp.full((R, 3 * LW), _BIG_KEY, jnp.int32)
    bu = bu_ref[...]
    lane = lax.broadcasted_iota(jnp.int32, (R, block_c), 1)

    def sweep1(j, _):
        @pl.when(jnp.logical_and(j >= jlo, j < jhi))
        def _():
            d2 = jnp.zeros((R, block_c), jnp.float32)
            for c in range(3):
                diff = pu_ref[:, c:c + 1] - posb_ref[j, c:c + 1, :]
                d2 = d2 + diff * diff
            key = (lax.bitcast_convert_type(d2, jnp.int32) & _MASK12) \
                | (lane + j * block_c)
            key = jnp.where(bu == bcb_ref[j], key, _BIG_KEY)
            u1 = m_ref[:, 0 * LW:1 * LW]
            u2 = m_ref[:, 1 * LW:2 * LW]
            u3 = m_ref[:, 2 * LW:3 * LW]
            for s in range(block_c // LW):
                v = key[:, s * LW:(s + 1) * LW]
                t1 = jnp.maximum(u1, v)
                u1 = jnp.minimum(u1, v)
                t2 = jnp.maximum(u2, t1)
                u2 = jnp.minimum(u2, t1)
                u3 = jnp.minimum(u3, t2)
            m_ref[:, 0 * LW:1 * LW] = u1
            m_ref[:, 1 * LW:2 * LW] = u2
            m_ref[:, 2 * LW:3 * LW] = u3
        return 0

    lax.fori_loop(0, n_blocks, sweep1, 0, unroll=True)

    u1 = m_ref[:, 0 * 128:1 * 128]
    u2 = m_ref[:, 1 * 128:2 * 128]
    u3 = m_ref[:, 2 * 128:3 * 128]
    ms = []
    for _ in range(3):
        mk = jnp.min(u1, axis=1, keepdims=True)
        sel = u1 == mk
        u1 = jnp.where(sel, u2, u1)
        u2 = jnp.where(sel, u3, u2)
        u3 = jnp.where(sel, _BIG_KEY, u3)
        ms.append(mk)

    ws = []
    den = jnp.zeros((R, 1), jnp.float32)
    for mk in ms:
        dk = lax.bitcast_convert_type(mk & _MASK12, jnp.float32)
        wk = 1.0 / jnp.maximum(dk, 1e-16)
        den = den + wk
        ws.append(wk)
    for k in range(3):
        idx_ref[:, k:k + 1] = ms[k] & 4095
        w_ref[:, k * 16:(k + 1) * 16] = jnp.broadcast_to(ws[k] / den, (R, 16))


def _knn(pos_up, batch_up, pos, batch, row_tile, block_c):
    n_up = pos_up.shape[0]
    n_coarse = pos.shape[0]
    n_blocks = n_coarse // block_c
    n_tiles = n_up // row_tile

    bu_i = batch_up.reshape(n_tiles, row_tile)
    lo = jnp.searchsorted(batch, bu_i[:, 0], side="left")
    hi = jnp.searchsorted(batch, bu_i[:, -1], side="right")
    wins = jnp.stack([lo // block_c,
                      (hi + block_c - 1) // block_c], axis=1).astype(jnp.int32)

    posb = pos.T.reshape(3, n_blocks, block_c).transpose(1, 0, 2)
    bcb = batch.astype(jnp.int32).reshape(n_blocks, 1, block_c)
    bu2 = batch_up.astype(jnp.int32)[:, None]

    row_spec = lambda cols_: pl.BlockSpec((row_tile, cols_), lambda i: (i, 0))
    full = lambda a: pl.BlockSpec(a.shape, lambda i: (0,) * a.ndim)

    fn = pl.pallas_call(
        functools.partial(_knn_body, n_blocks=n_blocks, block_c=block_c),
        grid=(n_tiles,),
        in_specs=[
            pl.BlockSpec((1, 1, 2), lambda i: (i, 0, 0),
                         memory_space=pltpu.SMEM),
            row_spec(3),
            row_spec(1),
            full(posb),
            full(bcb),
        ],
        out_specs=(row_spec(3), row_spec(48)),
        out_shape=(jax.ShapeDtypeStruct((n_up, 3), jnp.int32),
                   jax.ShapeDtypeStruct((n_up, 48), jnp.float32)),
        scratch_shapes=[pltpu.VMEM((row_tile, 3 * 128), jnp.int32)],
    )
    return fn(wins[:, None, :], pos_up, bu2, posb, bcb)


def _sc_interp(x, idx_t, w_t):
    n_up = idx_t.shape[1]
    c_dim = x.shape[1]
    info = plsc.get_sparse_core_info()
    nw = info.num_cores * info.num_subcores
    b_per_w = n_up // nw
    chunk = 64
    n_chunks = b_per_w // chunk
    mesh = plsc.VectorSubcoreMesh(core_axis_name="c", subcore_axis_name="s")

    @functools.partial(
        pl.kernel, mesh=mesh,
        out_type=jax.ShapeDtypeStruct((n_up, c_dim), jnp.float32),
        scratch_types=[
            pltpu.VMEM((chunk,), jnp.int32),
            pltpu.VMEM((chunk,), jnp.int32),
            pltpu.VMEM((chunk,), jnp.int32),
            pltpu.VMEM((chunk, 48), jnp.float32),
            pltpu.VMEM((chunk, c_dim), jnp.float32),
            pltpu.VMEM((chunk, c_dim), jnp.float32),
            pltpu.VMEM((chunk, c_dim), jnp.float32),
            pltpu.VMEM((chunk, c_dim), jnp.float32),
            pltpu.SemaphoreType.DMA,
        ])
    def body(x_hbm, idx_hbm, w_hbm, out_hbm,
             i0_v, i1_v, i2_v, w_v, r0_v, r1_v, r2_v, out_v, sem):
        wid = lax.axis_index("s") * info.num_cores + lax.axis_index("c")
        base = wid * b_per_w
        for ci in range(n_chunks):
            cbase = base + ci * chunk
            pltpu.sync_copy(idx_hbm.at[0, pl.ds(cbase, chunk)], i0_v)
            pltpu.sync_copy(idx_hbm.at[1, pl.ds(cbase, chunk)], i1_v)
            pltpu.sync_copy(idx_hbm.at[2, pl.ds(cbase, chunk)], i2_v)
            pltpu.sync_copy(w_hbm.at[pl.ds(cbase, chunk)], w_v)
            d0 = pltpu.async_copy(x_hbm.at[i0_v], r0_v, sem)
            d1 = pltpu.async_copy(x_hbm.at[i1_v], r1_v, sem)
            d2 = pltpu.async_copy(x_hbm.at[i2_v], r2_v, sem)
            d0.wait()
            d1.wait()
            d2.wait()

            def row_body(r, _):
                w0 = w_v[r, pl.ds(0, 16)]
                w1 = w_v[r, pl.ds(16, 16)]
                w2 = w_v[r, pl.ds(32, 16)]
                for t in range(c_dim // 16):
                    sl = pl.ds(t * 16, 16)
                    out_v[r, sl] = (w0 * r0_v[r, sl] + w1 * r1_v[r, sl]
                                    + w2 * r2_v[r, sl])
                return 0

            lax.fori_loop(0, chunk, row_body, 0)
            pltpu.sync_copy(out_v, out_hbm.at[pl.ds(cbase, chunk)])

    return body(x, idx_t, w_t)


def _mlp_body(up_ref, xs_ref, w_top_ref, w_bot_ref, b_up_ref,
              wa_ref, ba_ref, wb_ref, bb_ref, out_ref):
    xi = jnp.dot(xs_ref[...], w_top_ref[...], preferred_element_type=jnp.float32)
    xi = xi + jnp.dot(up_ref[...], w_bot_ref[...],
                      preferred_element_type=jnp.float32)
    xi = jax.nn.relu(xi + b_up_ref[...])
    h = jax.nn.relu(jnp.dot(xi, wa_ref[...], preferred_element_type=jnp.float32)
                    + ba_ref[...])
    out_ref[...] = xi + jnp.dot(h, wb_ref[...],
                                preferred_element_type=jnp.float32) + bb_ref[...]


def _mlp(up, x_skip, W_up, b_up, Wa, ba, Wb, bb, row_tile):
    n_up, c_in = up.shape
    c_skip = x_skip.shape[1]
    c_out = Wa.shape[0]
    w_top = W_up[:c_skip]
    w_bot = W_up[c_skip:]
    row_spec = lambda cols_: pl.BlockSpec((row_tile, cols_), lambda i: (i, 0))
    full = lambda a: pl.BlockSpec(a.shape, lambda i: (0,) * a.ndim)
    fn = pl.pallas_call(
        _mlp_body,
        grid=(n_up // row_tile,),
        in_specs=[
            row_spec(c_in), row_spec(c_skip),
            full(w_top), full(w_bot), full(b_up[None, :]),
            full(Wa), full(ba[None, :]),
            full(Wb), full(bb[None, :]),
        ],
        out_specs=row_spec(c_out),
        out_shape=jax.ShapeDtypeStruct((n_up, c_out), jnp.float32),
    )
    return fn(up, x_skip, w_top, w_bot, b_up[None, :],
              Wa, ba[None, :], Wb, bb[None, :])


def kernel(pos0, pos1, pos2, x0, x1, x2, batch0, batch1, batch2,
           W_up1, b_up1, W_res1a, b_res1a, W_res1b, b_res1b,
           W_up2, b_up2, W_res2a, b_res2a, W_res2b, b_res2b):
    xi1 = _level(pos1, batch1, pos0, batch0, x0, x1,
                 W_up1, b_up1, W_res1a, b_res1a, W_res1b, b_res1b,
                 row_tile=512, block_c=256)
    idx2, w2 = _knn(pos2, batch2, pos1, batch1, row_tile=512, block_c=512)
    up2 = _sc_interp(xi1, idx2.T, w2)
    xi2 = _mlp(up2, x2, W_up2, b_up2, W_res2a, b_res2a, W_res2b, b_res2b,
               row_tile=512)
    return xi2


# speedup vs baseline: 1.0141x; 1.0141x over previous
import functools

import jax
import jax.numpy as jnp
from jax import lax
from jax.experimental import pallas as pl
from jax.experimental.pallas import tpu as pltpu
from jax.experimental.pallas import tpu_sc as plsc

_BIG_KEY = 0x7F7FFFFF
_MASK12 = ~4095


def _level_body(win_ref, pu_ref, bu_ref, posb_ref, bcb_ref, xb_ref, xs_ref,
                w_top_ref, w_bot_ref, b_up_ref, wa_ref, ba_ref,
                wb_ref, bb_ref, out_ref, m_ref, key_ref, up_ref,
                *, n_blocks, block_c):
    R = pu_ref.shape[0]
    jlo = win_ref[0, 0, 0]
    jhi = win_ref[0, 0, 1]

    LW = 128
    m_ref[...] = jnp.full((R, 3 * LW), _BIG_KEY, jnp.int32)
    up_ref[...] = jnp.zeros_like(up_ref)

    bu = bu_ref[...]
    lane = lax.broadcasted_iota(jnp.int32, (R, block_c), 1)

    def sweep1(j, _):
        @pl.when(jnp.logical_and(j >= jlo, j < jhi))
        def _():
            d2 = jnp.zeros((R, block_c), jnp.float32)
            for c in range(3):
                diff = pu_ref[:, c:c + 1] - posb_ref[j, c:c + 1, :]
                d2 = d2 + diff * diff
            key = (lax.bitcast_convert_type(d2, jnp.int32) & _MASK12) \
                | (lane + j * block_c)
            key = jnp.where(bu == bcb_ref[j], key, _BIG_KEY)
            key_ref[j] = key
            u1 = m_ref[:, 0 * LW:1 * LW]
            u2 = m_ref[:, 1 * LW:2 * LW]
            u3 = m_ref[:, 2 * LW:3 * LW]
            for s in range(block_c // LW):
                v = key[:, s * LW:(s + 1) * LW]
                t1 = jnp.maximum(u1, v)
                u1 = jnp.minimum(u1, v)
                t2 = jnp.maximum(u2, t1)
                u2 = jnp.minimum(u2, t1)
                u3 = jnp.minimum(u3, t2)
            m_ref[:, 0 * LW:1 * LW] = u1
            m_ref[:, 1 * LW:2 * LW] = u2
            m_ref[:, 2 * LW:3 * LW] = u3
        return 0

    lax.fori_loop(0, n_blocks, sweep1, 0, unroll=True)

    u1 = m_ref[:, 0 * LW:1 * LW]
    u2 = m_ref[:, 1 * LW:2 * LW]
    u3 = m_ref[:, 2 * LW:3 * LW]
    ms = []
    for _ in range(3):
        mk = jnp.min(u1, axis=1, keepdims=True)
        sel = u1 == mk
        u1 = jnp.where(sel, u2, u1)
        u2 = jnp.where(sel, u3, u2)
        u3 = jnp.where(sel, _BIG_KEY, u3)
        ms.append(mk)
    m3 = ms[2]

    def sweep2(j, _):
        @pl.when(jnp.logical_and(j >= jlo, j < jhi))
        def _():
            key = key_ref[j]
            d2q = lax.bitcast_convert_type(key & _MASK12, jnp.float32)
            w = 1.0 / jnp.maximum(d2q, 1e-16)
            wmat = jnp.where(key <= m3, w, 0.0)
            up_ref[...] += jnp.dot(wmat, xb_ref[j],
                                   preferred_element_type=jnp.float32)
        return 0

    lax.fori_loop(0, n_blocks, sweep2, 0, unroll=True)

    den = jnp.zeros((R, 1), jnp.float32)
    for mk in ms:
        dk = lax.bitcast_convert_type(mk & _MASK12, jnp.float32)
        den = den + 1.0 / jnp.maximum(dk, 1e-16)
    up = up_ref[...] / den

    xi = jnp.dot(xs_ref[...], w_top_ref[...], preferred_element_type=jnp.float32)
    xi = xi + jnp.dot(up, w_bot_ref[...], preferred_element_type=jnp.float32)
    xi = jax.nn.relu(xi + b_up_ref[...])
    h = jax.nn.relu(jnp.dot(xi, wa_ref[...], preferred_element_type=jnp.float32)
                    + ba_ref[...])
    out_ref[...] = xi + jnp.dot(h, wb_ref[...],
                                preferred_element_type=jnp.float32) + bb_ref[...]


def _level(pos_up, batch_up, pos, batch, x, x_skip,
           W_up, b_up, Wa, ba, Wb, bb, row_tile, block_c):
    n_up = pos_up.shape[0]
    n_coarse = pos.shape[0]
    c_skip = x_skip.shape[1]
    c_out = Wa.shape[0]
    n_blocks = n_coarse // block_c
    n_tiles = n_up // row_tile

    bu_i = batch_up.reshape(n_tiles, row_tile)
    lo = jnp.searchsorted(batch, bu_i[:, 0], side="left")
    hi = jnp.searchsorted(batch, bu_i[:, -1], side="right")
    wins = jnp.stack([lo // block_c,
                      (hi + block_c - 1) // block_c], axis=1).astype(jnp.int32)

    posb = pos.T.reshape(3, n_blocks, block_c).transpose(1, 0, 2)
    bcb = batch.astype(jnp.int32).reshape(n_blocks, 1, block_c)
    xb = x.reshape(n_blocks, block_c, x.shape[1])
    bu2 = batch_up.astype(jnp.int32)[:, None]
    w_top = W_up[:c_skip]
    w_bot = W_up[c_skip:]

    row_spec = lambda cols_: pl.BlockSpec((row_tile, cols_), lambda i: (i, 0))
    full = lambda a: pl.BlockSpec(a.shape, lambda i: (0,) * a.ndim)

    fn = pl.pallas_call(
        functools.partial(_level_body, n_blocks=n_blocks, block_c=block_c),
        grid=(n_tiles,),
        in_specs=[
            pl.BlockSpec((1, 1, 2), lambda i: (i, 0, 0),
                         memory_space=pltpu.SMEM),
            row_spec(3),
            row_spec(1),
            full(posb),
            full(bcb),
            full(xb),
            row_spec(c_skip),
            full(w_top), full(w_bot), full(b_up[None, :]),
            full(Wa), full(ba[None, :]),
            full(Wb), full(bb[None, :]),
        ],
        out_specs=row_spec(c_out),
        out_shape=jax.ShapeDtypeStruct((n_up, c_out), jnp.float32),
        scratch_shapes=[
            pltpu.VMEM((row_tile, 3 * 128), jnp.int32),
            pltpu.VMEM((n_blocks, row_tile, block_c), jnp.int32),
            pltpu.VMEM((row_tile, x.shape[1]), jnp.float32),
        ],
    )
    return fn(wins[:, None, :], pos_up, bu2, posb, bcb, xb, x_skip,
              w_top, w_bot, b_up[None, :], Wa, ba[None, :], Wb, bb[None, :])


def _knn_body(win_ref, pu_ref, bu_ref, posb_ref, bcb_ref,
              idx_ref, w_ref, m_ref, *, n_blocks, block_c):
    R = pu_ref.shape[0]
    jlo = win_ref[0, 0, 0]
    jhi = win_ref[0, 0, 1]

    LW = 128
    m_ref[...] = jnp.full((R, 3 * LW), _BIG_KEY, jnp.int32)
    bu = bu_ref[...]
    lane = lax.broadcasted_iota(jnp.int32, (R, block_c), 1)

    def sweep1(j, _):
        @pl.when(jnp.logical_and(j >= jlo, j < jhi))
        def _():
            d2 = jnp.zeros((R, block_c), jnp.float32)
            for c in range(3):
                diff = pu_ref[:, c:c + 1] - posb_ref[j, c:c + 1, :]
                d2 = d2 + diff * diff
            key = (lax.bitcast_convert_type(d2, jnp.int32) & _MASK12) \
                | (lane + j * block_c)
            key = jnp.where(bu == bcb_ref[j], key, _BIG_KEY)
            u1 = m_ref[:, 0 * LW:1 * LW]
            u2 = m_ref[:, 1 * LW:2 * LW]
            u3 = m_ref[:, 2 * LW:3 * LW]
            for s in range(block_c // LW):
                v = key[:, s * LW:(s + 1) * LW]
                t1 = jnp.maximum(u1, v)
                u1 = jnp.minimum(u1, v)
                t2 = jnp.maximum(u2, t1)
                u2 = jnp.minimum(u2, t1)
                u3 = jnp.minimum(u3, t2)
            m_ref[:, 0 * LW:1 * LW] = u1
            m_ref[:, 1 * LW:2 * LW] = u2
            m_ref[:, 2 * LW:3 * LW] = u3
        return 0

    lax.fori_loop(0, n_blocks, sweep1, 0, unroll=True)

    u1 = m_ref[:, 0 * 128:1 * 128]
    u2 = m_ref[:, 1 * 128:2 * 128]
    u3 = m_ref[:, 2 * 128:3 * 128]
    ms = []
    for _ in range(3):
        mk = jnp.min(u1, axis=1, keepdims=True)
        sel = u1 == mk
        u1 = jnp.where(sel, u2, u1)
        u2 = jnp.where(sel, u3, u2)
        u3 = jnp.where(sel, _BIG_KEY, u3)
        ms.append(mk)

    ws = []
    den = jnp.zeros((R, 1), jnp.float32)
    for mk in ms:
        dk = lax.bitcast_convert_type(mk & _MASK12, jnp.float32)
        wk = 1.0 / jnp.maximum(dk, 1e-16)
        den = den + wk
        ws.append(wk)
    for k in range(3):
        idx_ref[:, k:k + 1] = ms[k] & 4095
        w_ref[:, k * 16:(k + 1) * 16] = jnp.broadcast_to(ws[k] / den, (R, 16))


def _knn(pos_up, batch_up, pos, batch, row_tile, block_c):
    n_up = pos_up.shape[0]
    n_coarse = pos.shape[0]
    n_blocks = n_coarse // block_c
    n_tiles = n_up // row_tile

    bu_i = batch_up.reshape(n_tiles, row_tile)
    lo = jnp.searchsorted(batch, bu_i[:, 0], side="left")
    hi = jnp.searchsorted(batch, bu_i[:, -1], side="right")
    wins = jnp.stack([lo // block_c,
                      (hi + block_c - 1) // block_c], axis=1).astype(jnp.int32)

    posb = pos.T.reshape(3, n_blocks, block_c).transpose(1, 0, 2)
    bcb = batch.astype(jnp.int32).reshape(n_blocks, 1, block_c)
    bu2 = batch_up.astype(jnp.int32)[:, None]

    row_spec = lambda cols_: pl.BlockSpec((row_tile, cols_), lambda i: (i, 0))
    full = lambda a: pl.BlockSpec(a.shape, lambda i: (0,) * a.ndim)

    fn = pl.pallas_call(
        functools.partial(_knn_body, n_blocks=n_blocks, block_c=block_c),
        grid=(n_tiles,),
        in_specs=[
            pl.BlockSpec((1, 1, 2), lambda i: (i, 0, 0),
                         memory_space=pltpu.SMEM),
            row_spec(3),
            row_spec(1),
            full(posb),
            full(bcb),
        ],
        out_specs=(row_spec(3), row_spec(48)),
        out_shape=(jax.ShapeDtypeStruct((n_up, 3), jnp.int32),
                   jax.ShapeDtypeStruct((n_up, 48), jnp.float32)),
        scratch_shapes=[pltpu.VMEM((row_tile, 3 * 128), jnp.int32)],
    )
    return fn(wins[:, None, :], pos_up, bu2, posb, bcb)


def _sc_interp(x, idx_t, w_t):
    n_up = idx_t.shape[1]
    c_dim = x.shape[1]
    info = plsc.get_sparse_core_info()
    nw = info.num_cores * info.num_subcores
    b_per_w = n_up // nw
    chunk = 32
    n_chunks = b_per_w // chunk
    mesh = plsc.VectorSubcoreMesh(core_axis_name="c", subcore_axis_name="s")

    buf_types = [
        pltpu.VMEM((chunk,), jnp.int32),
        pltpu.VMEM((chunk,), jnp.int32),
        pltpu.VMEM((chunk,), jnp.int32),
        pltpu.VMEM((chunk, 48), jnp.float32),
        pltpu.VMEM((chunk, c_dim), jnp.float32),
        pltpu.VMEM((chunk, c_dim), jnp.float32),
        pltpu.VMEM((chunk, c_dim), jnp.float32),
        pltpu.VMEM((chunk, c_dim), jnp.float32),
        pltpu.SemaphoreType.DMA,
    ]

    @functools.partial(
        pl.kernel, mesh=mesh,
        out_type=jax.ShapeDtypeStruct((n_up, c_dim), jnp.float32),
        scratch_types=buf_types + buf_types)
    def body(x_hbm, idx_hbm, w_hbm, out_hbm, *scr):
        wid = lax.axis_index("s") * info.num_cores + lax.axis_index("c")
        base = wid * b_per_w
        bufs = (scr[:9], scr[9:])
        pend = {}

        def fire(ci):
            i0_v, i1_v, i2_v, w_v, r0_v, r1_v, r2_v, _, sem = bufs[ci % 2]
            cbase = base + ci * chunk
            pltpu.sync_copy(idx_hbm.at[0, pl.ds(cbase, chunk)], i0_v)
            pltpu.sync_copy(idx_hbm.at[1, pl.ds(cbase, chunk)], i1_v)
            pltpu.sync_copy(idx_hbm.at[2, pl.ds(cbase, chunk)], i2_v)
            pltpu.sync_copy(w_hbm.at[pl.ds(cbase, chunk)], w_v)
            pend[ci] = [pltpu.async_copy(x_hbm.at[i0_v], r0_v, sem),
                        pltpu.async_copy(x_hbm.at[i1_v], r1_v, sem),
                        pltpu.async_copy(x_hbm.at[i2_v], r2_v, sem)]

        fire(0)
        for ci in range(n_chunks):
            if ci + 1 < n_chunks:
                fire(ci + 1)
            _, _, _, w_v, r0_v, r1_v, r2_v, out_v, _ = bufs[ci % 2]
            for d in pend.pop(ci):
                d.wait()

            def row_body(r, _, w_v=w_v, r0_v=r0_v, r1_v=r1_v, r2_v=r2_v,
                         out_v=out_v):
                w0 = w_v[r, pl.ds(0, 16)]
                w1 = w_v[r, pl.ds(16, 16)]
                w2 = w_v[r, pl.ds(32, 16)]
                for t in range(c_dim // 16):
                    sl = pl.ds(t * 16, 16)
                    out_v[r, sl] = (w0 * r0_v[r, sl] + w1 * r1_v[r, sl]
                                    + w2 * r2_v[r, sl])
                return 0

            lax.fori_loop(0, chunk, row_body, 0)
            pltpu.sync_copy(out_v, out_hbm.at[pl.ds(base + ci * chunk, chunk)])

    return body(x, idx_t, w_t)


def _mlp_body(up_ref, xs_ref, w_top_ref, w_bot_ref, b_up_ref,
              wa_ref, ba_ref, wb_ref, bb_ref, out_ref):
    xi = jnp.dot(xs_ref[...], w_top_ref[...], preferred_element_type=jnp.float32)
    xi = xi + jnp.dot(up_ref[...], w_bot_ref[...],
                      preferred_element_type=jnp.float32)
    xi = jax.nn.relu(xi + b_up_ref[...])
    h = jax.nn.relu(jnp.dot(xi, wa_ref[...], preferred_element_type=jnp.float32)
                    + ba_ref[...])
    out_ref[...] = xi + jnp.dot(h, wb_ref[...],
                                preferred_element_type=jnp.float32) + bb_ref[...]


def _mlp(up, x_skip, W_up, b_up, Wa, ba, Wb, bb, row_tile):
    n_up, c_in = up.shape
    c_skip = x_skip.shape[1]
    c_out = Wa.shape[0]
    w_top = W_up[:c_skip]
    w_bot = W_up[c_skip:]
    row_spec = lambda cols_: pl.BlockSpec((row_tile, cols_), lambda i: (i, 0))
    full = lambda a: pl.BlockSpec(a.shape, lambda i: (0,) * a.ndim)
    fn = pl.pallas_call(
        _mlp_body,
        grid=(n_up // row_tile,),
        in_specs=[
            row_spec(c_in), row_spec(c_skip),
            full(w_top), full(w_bot), full(b_up[None, :]),
            full(Wa), full(ba[None, :]),
            full(Wb), full(bb[None, :]),
        ],
        out_specs=row_spec(c_out),
        out_shape=jax.ShapeDtypeStruct((n_up, c_out), jnp.float32),
    )
    return fn(up, x_skip, w_top, w_bot, b_up[None, :],
              Wa, ba[None, :], Wb, bb[None, :])


def kernel(pos0, pos1, pos2, x0, x1, x2, batch0, batch1, batch2,
           W_up1, b_up1, W_res1a, b_res1a, W_res1b, b_res1b,
           W_up2, b_up2, W_res2a, b_res2a, W_res2b, b_res2b):
    xi1 = _level(pos1, batch1, pos0, batch0, x0, x1,
                 W_up1, b_up1, W_res1a, b_res1a, W_res1b, b_res1b,
                 row_tile=512, block_c=256)
    idx2, w2 = _knn(pos2, batch2, pos1, batch1, row_tile=512, block_c=512)
    up2 = _sc_interp(xi1, idx2.T, w2)
    xi2 = _mlp(up2, x2, W_up2, b_up2, W_res2a, b_res2a, W_res2b, b_res2b,
               row_tile=512)
    return xi2
